# Initial kernel scaffold; baseline (speedup 1.0000x reference)
import jax
import jax.numpy as jnp
from jax.experimental import pallas as pl


def kernel(pts_before, kp_before, kp_warped_pred, pose_gt, overlap_weights):
    raise NotImplementedError("write your pallas kernel here")


# speedup vs baseline: 11.6571x; 11.6571x over previous
import functools

import jax
import jax.numpy as jnp
from jax.experimental import pallas as pl
from jax.experimental.pallas import tpu as pltpu

_EPS = 1e-06
_K = 50
_B = 4
_M = 512
_N = 16384
_CHUNK = 128
_C = _M // _CHUNK


def _tc_kernel(pts_t_ref, nodes_ref, kpw_ref, pose_ref, ow_ref, out_ref, acc_ref):
    b = pl.program_id(0)
    c = pl.program_id(1)

    @pl.when(jnp.logical_and(b == 0, c == 0))
    def _init():
        acc_ref[0] = 0.0
        acc_ref[1] = 0.0

    pts_t = pts_t_ref[0]
    nodes = nodes_ref[0]

    ndp = jax.lax.dot_general(nodes, pts_t, (((1,), (0,)), ((), ())),
                              preferred_element_type=jnp.float32)
    psq = jnp.sum(pts_t * pts_t, axis=0, keepdims=True)
    nsq = jnp.sum(nodes * nodes, axis=1, keepdims=True)
    d = (nsq + psq) - 2.0 * ndp

    bits = jax.lax.bitcast_convert_type(d, jnp.int32)
    skey = jnp.where(bits >= 0, bits, bits ^ jnp.int32(0x7FFFFFFF))

    def search_step(i, prefix):
        bit = (jnp.uint32(31) - i.astype(jnp.uint32))
        probe = prefix + (jnp.uint32(1) << bit)
        sprobe = jax.lax.bitcast_convert_type(
            probe ^ jnp.uint32(0x80000000), jnp.int32)
        cnt = jnp.sum((skey < sprobe[:, None]).astype(jnp.int32), axis=1)
        return jnp.where(cnt >= _K, prefix, probe)

    prefix0 = jnp.zeros((_CHUNK,), dtype=jnp.uint32)
    t_u = jax.lax.fori_loop(0, 32, search_step, prefix0)
    t_s = jax.lax.bitcast_convert_type(t_u ^ jnp.uint32(0x80000000), jnp.int32)

    lt = skey < t_s[:, None]
    eq = skey == t_s[:, None]
    cnt_lt = jnp.sum(lt.astype(jnp.int32), axis=1)
    t_cnt = jnp.sum(eq.astype(jnp.int32), axis=1)
    r = (_K - cnt_lt).astype(jnp.float32)
    frac = r / t_cnt.astype(jnp.float32)
    w = lt.astype(jnp.float32) + frac[:, None] * eq.astype(jnp.float32)

    sums = jax.lax.dot_general(w, pts_t, (((1,), (1,)), ((), ())),
                               preferred_element_type=jnp.float32)
    mean = sums * (1.0 / _K)

    pose = pose_ref[0]
    rot = pose[:3, :3]
    trans = pose[:3, 3]
    mt = jax.lax.dot_general(mean, rot, (((1,), (1,)), ((), ())),
                             preferred_element_type=jnp.float32) + trans[None, :]
    corr = jnp.sum(jnp.abs(mt - kpw_ref[0]), axis=1)
    ow = ow_ref[0, 0]

    acc_ref[0] += jnp.sum(ow * corr)
    acc_ref[1] += jnp.sum(ow)

    @pl.when(jnp.logical_and(b == _B - 1, c == _C - 1))
    def _fin():
        out_ref[...] = jnp.full((1, 1), acc_ref[0] / jnp.maximum(acc_ref[1], _EPS),
                                dtype=jnp.float32)


@jax.jit
def kernel(pts_before, kp_before, kp_warped_pred, pose_gt, overlap_weights):
    pts_t = jnp.swapaxes(pts_before, 1, 2)
    ow3 = overlap_weights.reshape(_B * _C, 1, _CHUNK)

    out = pl.pallas_call(
        _tc_kernel,
        grid=(_B, _C),
        in_specs=[
            pl.BlockSpec((1, 3, _N), lambda b, c: (b, 0, 0)),
            pl.BlockSpec((1, _CHUNK, 3), lambda b, c: (b, c, 0)),
            pl.BlockSpec((1, _CHUNK, 3), lambda b, c: (b, c, 0)),
            pl.BlockSpec((1, 4, 4), lambda b, c: (b, 0, 0)),
            pl.BlockSpec((1, 1, _CHUNK), lambda b, c: (b * _C + c, 0, 0)),
        ],
        out_specs=pl.BlockSpec((1, 1), lambda b, c: (0, 0)),
        out_shape=jax.ShapeDtypeStruct((1, 1), jnp.float32),
        scratch_shapes=[pltpu.SMEM((2,), jnp.float32)],
    )(pts_t, kp_before, kp_warped_pred, pose_gt, ow3)
    return jnp.reshape(out, ())


# speedup vs baseline: 12.4642x; 1.0692x over previous
import functools

import jax
import jax.numpy as jnp
from jax import lax
from jax.experimental import pallas as pl
from jax.experimental.pallas import tpu as pltpu
from jax.experimental.pallas import tpu_sc as plsc

_EPS = 1e-06
_K = 50
_B = 4
_M = 512
_N = 16384
_L = 16
_NC = 2
_NS = 16
_NW = _NC * _NS
_TPB = _NW // _B
_NPT = _M // _TPB
_NG = _NPT // _L
_CAP = 256
_KCHK = 32
_MAXFIN = 0x7F7FFFFF


def _sc_body(xs_h, ys_h, zs_h, nd_h, kw_h, ps_h, ow_h, out_h,
             xs, ys, zs, nd, kw, ps, ow, bkey, bidx, outv):
    cid = lax.axis_index("c")
    sid = lax.axis_index("s")
    wid = sid * _NC + cid
    batch = wid // _TPB
    s8 = wid % _TPB

    pltpu.sync_copy(xs_h.at[batch], xs)
    pltpu.sync_copy(ys_h.at[batch], ys)
    pltpu.sync_copy(zs_h.at[batch], zs)
    pltpu.sync_copy(nd_h.at[batch], nd)
    pltpu.sync_copy(kw_h.at[batch], kw)
    pltpu.sync_copy(ps_h.at[batch], ps)
    pltpu.sync_copy(ow_h.at[batch], ow)

    col = lax.iota(jnp.int32, _L)
    base = s8 * _NPT

    nx = [nd[0, pl.ds(base + g * _L, _L)] for g in range(_NG)]
    ny = [nd[1, pl.ds(base + g * _L, _L)] for g in range(_NG)]
    nz = [nd[2, pl.ds(base + g * _L, _L)] for g in range(_NG)]
    colg = [col + g * _L for g in range(_NG)]
    _ROWW = _NG * _L

    zero16 = jnp.zeros((_L,), dtype=jnp.int32)
    one16 = jnp.full((_L,), 1, dtype=jnp.int32)

    def prune_group(g, cnt, t_cur):
        cmax = jnp.max(cnt)

        def do(args):
            cnt, t_cur = args
            f = bkey[pl.ds(g * _L, _L)]
            fmin = f
            for k in range(1, _K):
                row = bkey[pl.ds(k * _ROWW + g * _L, _L)]
                f = jnp.maximum(f, row)
                fmin = jnp.minimum(fmin, row)

            def ladder(lo, hi):
                step = lax.shift_right_arithmetic(hi - lo, 3)
                probes = [lo + i * step for i in range(1, 8)]

                def count_row(k, cs):
                    row = bkey[pl.ds(k * _ROWW + g * _L, _L)]
                    valid = k < cnt
                    return tuple(
                        c + jnp.where(jnp.logical_and(row < p, valid), one16, zero16)
                        for c, p in zip(cs, probes))

                counts = lax.fori_loop(0, cmax, count_row,
                                       tuple(zero16 for _ in probes))
                t_hi = hi
                for c, p in zip(reversed(counts), reversed(probes)):
                    t_hi = jnp.where(c >= _K, p, t_hi)
                t_lo = lo
                for c, p in zip(counts, probes):
                    t_lo = jnp.where(c < _K, p, t_lo)
                return t_hi, t_lo

            t1, lo1 = ladder(fmin, f)
            t_new, _ = ladder(lo1, t1)
            t_new = jnp.where(cnt >= _K, t_new, t_cur)

            def compact_row(k, ncnt):
                rk = bkey[pl.ds(k * _ROWW + g * _L, _L)]
                ri = bidx[pl.ds(k * _ROWW + g * _L, _L)]
                keep = jnp.logical_and(rk <= t_new, k < cnt)
                nc = jnp.minimum(ncnt, _CAP - 1) * _ROWW + colg[g]
                plsc.store_scatter(bkey, [nc], rk, mask=keep)
                plsc.store_scatter(bidx, [nc], ri, mask=keep)
                return ncnt + jnp.where(keep, one16, zero16)

            new_cnt = lax.fori_loop(0, cmax, compact_row, zero16)
            return new_cnt, t_new

        return lax.cond(cmax >= _K, do, lambda a: a, (cnt, t_cur))

    def prune_all(carry):
        cnts, ts = carry
        new = [prune_group(g, cnts[g], ts[g]) for g in range(_NG)]
        return tuple(n[0] for n in new), tuple(n[1] for n in new)

    nsq = [(nx[g] * nx[g] + ny[g] * ny[g]) + nz[g] * nz[g] for g in range(_NG)]

    def chunk_step(ci, carry):
        cnts, ts = carry
        j0 = ci * _L
        px16 = xs[pl.ds(j0, _L)]
        py16 = ys[pl.ds(j0, _L)]
        pz16 = zs[pl.ds(j0, _L)]
        psq16 = (px16 * px16 + py16 * py16) + pz16 * pz16
        cnts = list(cnts)
        for i in range(_L):
            px = jnp.full((_L,), px16[i])
            py = jnp.full((_L,), py16[i])
            pz = jnp.full((_L,), pz16[i])
            psq = jnp.full((_L,), psq16[i])
            jv = jnp.full((_L,), 1, dtype=jnp.int32) * (j0 + i)
            for g in range(_NG):
                ndp = nx[g] * px + ny[g] * py + nz[g] * pz
                d = (nsq[g] + psq) - 2.0 * ndp
                kb = lax.bitcast_convert_type(d, jnp.int32)
                m = kb < ts[g]
                cidx = jnp.minimum(cnts[g], _CAP - 1)
                fidx = cidx * _ROWW + colg[g]
                plsc.store_scatter(bkey, [fidx], kb, mask=m)
                plsc.store_scatter(bidx, [fidx], jv, mask=m)
                cnts[g] = cnts[g] + jnp.where(m, one16, zero16)
        cnts = tuple(cnts)

        def check(carry):
            cnts, ts = carry
            maxc = jnp.max(jnp.maximum(jnp.maximum(cnts[0], cnts[1]),
                                       jnp.maximum(cnts[2], cnts[3])))
            return lax.cond(maxc >= _CAP - _L, prune_all, lambda a: a,
                            (cnts, ts))

        return check((cnts, ts))

    init = (tuple(zero16 for _ in range(_NG)),
            tuple(jnp.full((_L,), _MAXFIN, dtype=jnp.int32) for _ in range(_NG)))
    cnts, ts = lax.fori_loop(0, _N // _L, chunk_step, init)
    cnts, ts = prune_all((cnts, ts))

    num = jnp.float32(0.0)
    den = jnp.float32(0.0)
    for g in range(_NG):
        cnt = cnts[g]
        cmax = jnp.max(cnt)

        def bit_step(b, prefix):
            probe = prefix + (jnp.int32(1) << (jnp.int32(30) - b))

            def row_cnt(k, c):
                valid = jnp.logical_and(bkey[pl.ds(k * _ROWW + g * _L, _L)] < probe, k < cnt)
                return c + jnp.where(valid, one16, zero16)

            c = lax.fori_loop(0, cmax, row_cnt, zero16)
            return jnp.where(c >= _K, prefix, probe)

        t50 = lax.fori_loop(0, 31, bit_step, zero16)

        def stats_row(k, cs):
            c_lt, c_eq = cs
            row = bkey[pl.ds(k * _ROWW + g * _L, _L)]
            valid = k < cnt
            c_lt = c_lt + jnp.where(jnp.logical_and(row < t50, valid), one16, zero16)
            c_eq = c_eq + jnp.where(jnp.logical_and(row == t50, valid), one16, zero16)
            return c_lt, c_eq

        c_lt, c_eq = lax.fori_loop(0, cmax, stats_row, (zero16, zero16))
        frac = (_K - c_lt).astype(jnp.float32) / jnp.maximum(c_eq, 1).astype(jnp.float32)

        def sum_row(k, acc):
            sx, sy, sz = acc
            row = bkey[pl.ds(k * _ROWW + g * _L, _L)]
            ri = bidx[pl.ds(k * _ROWW + g * _L, _L)]
            valid = k < cnt
            lt = jnp.logical_and(row < t50, valid)
            eq = jnp.logical_and(row == t50, valid)
            w = jnp.where(lt, 1.0, 0.0) + jnp.where(eq, frac, 0.0)
            ri = jnp.bitwise_and(ri, _N - 1)
            gx = plsc.load_gather(xs, [ri])
            gy = plsc.load_gather(ys, [ri])
            gz = plsc.load_gather(zs, [ri])
            return sx + w * gx, sy + w * gy, sz + w * gz

        fzero = jnp.zeros((_L,), dtype=jnp.float32)
        sx, sy, sz = lax.fori_loop(0, cmax, sum_row, (fzero, fzero, fzero))
        mx = sx * (1.0 / _K)
        my = sy * (1.0 / _K)
        mz = sz * (1.0 / _K)

        pv = ps[pl.ds(0, _L)]
        mtx = pv[0] * mx + pv[1] * my + pv[2] * mz + pv[3]
        mty = pv[4] * mx + pv[5] * my + pv[6] * mz + pv[7]
        mtz = pv[8] * mx + pv[9] * my + pv[10] * mz + pv[11]

        sl = pl.ds(base + g * _L, _L)
        corr = (jnp.abs(mtx - kw[0, sl]) + jnp.abs(mty - kw[1, sl])
                + jnp.abs(mtz - kw[2, sl]))
        oww = ow[sl]
        num = num + jnp.sum(oww * corr)
        den = den + jnp.sum(oww)

    lane = lax.iota(jnp.int32, _L)
    outv[...] = jnp.where(lane == 0, num, jnp.where(lane == 1, den, 0.0))
    pltpu.sync_copy(outv, out_h.at[wid])


_sc_call = functools.partial(
    pl.kernel,
    out_type=jax.ShapeDtypeStruct((_NW, _L), jnp.float32),
    mesh=plsc.VectorSubcoreMesh(core_axis_name="c", subcore_axis_name="s",
                                num_cores=_NC, num_subcores=_NS),
    compiler_params=pltpu.CompilerParams(needs_layout_passes=False),
    scratch_types=[
        pltpu.VMEM((_N,), jnp.float32),
        pltpu.VMEM((_N,), jnp.float32),
        pltpu.VMEM((_N,), jnp.float32),
        pltpu.VMEM((3, _M), jnp.float32),
        pltpu.VMEM((3, _M), jnp.float32),
        pltpu.VMEM((_L,), jnp.float32),
        pltpu.VMEM((_M,), jnp.float32),
        pltpu.VMEM((_CAP * _NG * _L,), jnp.int32),
        pltpu.VMEM((_CAP * _NG * _L,), jnp.int32),
        pltpu.VMEM((_L,), jnp.float32),
    ],
)(_sc_body)


@jax.jit
def kernel(pts_before, kp_before, kp_warped_pred, pose_gt, overlap_weights):
    xs = pts_before[:, :, 0]
    ys = pts_before[:, :, 1]
    zs = pts_before[:, :, 2]
    nd = jnp.swapaxes(kp_before, 1, 2)
    kw = jnp.swapaxes(kp_warped_pred, 1, 2)
    ps = pose_gt.reshape(_B, 16)
    parts = _sc_call(xs, ys, zs, nd, kw, ps, overlap_weights)
    num = jnp.sum(parts[:, 0])
    den = jnp.sum(parts[:, 1])
    return num / jnp.maximum(den, _EPS)


# speedup vs baseline: 21.3638x; 1.7140x over previous
import functools

import jax
import jax.numpy as jnp
from jax import lax
from jax.experimental import pallas as pl
from jax.experimental.pallas import tpu as pltpu
from jax.experimental.pallas import tpu_sc as plsc

_EPS = 1e-06
_K = 50
_B = 4
_M = 512
_N = 16384
_L = 16
_NC = 2
_NS = 16
_NW = _NC * _NS
_TPB = _NW // _B
_MSC = 256
_NPT = _MSC // _TPB
_NG = _NPT // _L
_MTC = _M - _MSC
_CHUNK = 128
_C = _MTC // _CHUNK
_CAP = 256
_KCHK = 32
_MAXFIN = 0x7F7FFFFF


def _sc_body(xs_h, ys_h, zs_h, nd_h, kw_h, ps_h, ow_h, out_h,
             xs, ys, zs, nd, kw, ps, ow, bkey, bidx, outv):
    cid = lax.axis_index("c")
    sid = lax.axis_index("s")
    wid = sid * _NC + cid
    batch = wid // _TPB
    s8 = wid % _TPB

    pltpu.sync_copy(xs_h.at[batch], xs)
    pltpu.sync_copy(ys_h.at[batch], ys)
    pltpu.sync_copy(zs_h.at[batch], zs)
    pltpu.sync_copy(nd_h.at[batch], nd)
    pltpu.sync_copy(kw_h.at[batch], kw)
    pltpu.sync_copy(ps_h.at[batch], ps)
    pltpu.sync_copy(ow_h.at[batch], ow)

    col = lax.iota(jnp.int32, _L)
    base = s8 * _NPT

    nx = [nd[0, pl.ds(base + g * _L, _L)] for g in range(_NG)]
    ny = [nd[1, pl.ds(base + g * _L, _L)] for g in range(_NG)]
    nz = [nd[2, pl.ds(base + g * _L, _L)] for g in range(_NG)]
    colg = [col + g * _L for g in range(_NG)]
    _ROWW = _NG * _L

    zero16 = jnp.zeros((_L,), dtype=jnp.int32)
    one16 = jnp.full((_L,), 1, dtype=jnp.int32)

    def prune_group(g, cnt, t_cur):
        cmax = jnp.max(cnt)

        def do(args):
            cnt, t_cur = args
            f = bkey[pl.ds(g * _L, _L)]
            fmin = f
            for k in range(1, _K):
                row = bkey[pl.ds(k * _ROWW + g * _L, _L)]
                f = jnp.maximum(f, row)
                fmin = jnp.minimum(fmin, row)

            def ladder(lo, hi):
                step = lax.shift_right_arithmetic(hi - lo, 3)
                probes = [lo + i * step for i in range(1, 8)]

                def count_row(k, cs):
                    row = bkey[pl.ds(k * _ROWW + g * _L, _L)]
                    valid = k < cnt
                    return tuple(
                        c + jnp.where(jnp.logical_and(row < p, valid), one16, zero16)
                        for c, p in zip(cs, probes))

                counts = lax.fori_loop(0, cmax, count_row,
                                       tuple(zero16 for _ in probes))
                t_hi = hi
                for c, p in zip(reversed(counts), reversed(probes)):
                    t_hi = jnp.where(c >= _K, p, t_hi)
                t_lo = lo
                for c, p in zip(counts, probes):
                    t_lo = jnp.where(c < _K, p, t_lo)
                return t_hi, t_lo

            t1, lo1 = ladder(fmin, f)
            t_new, _ = ladder(lo1, t1)
            t_new = jnp.where(cnt >= _K, t_new, t_cur)

            def compact_row(k, ncnt):
                rk = bkey[pl.ds(k * _ROWW + g * _L, _L)]
                ri = bidx[pl.ds(k * _ROWW + g * _L, _L)]
                keep = jnp.logical_and(rk <= t_new, k < cnt)
                nc = jnp.minimum(ncnt, _CAP - 1) * _ROWW + colg[g]
                plsc.store_scatter(bkey, [nc], rk, mask=keep)
                plsc.store_scatter(bidx, [nc], ri, mask=keep)
                return ncnt + jnp.where(keep, one16, zero16)

            new_cnt = lax.fori_loop(0, cmax, compact_row, zero16)
            return new_cnt, t_new

        return lax.cond(cmax >= _K, do, lambda a: a, (cnt, t_cur))

    def prune_all(carry):
        cnts, ts = carry
        new = [prune_group(g, cnts[g], ts[g]) for g in range(_NG)]
        return tuple(n[0] for n in new), tuple(n[1] for n in new)

    nsq = [(nx[g] * nx[g] + ny[g] * ny[g]) + nz[g] * nz[g] for g in range(_NG)]

    def chunk_step(ci, carry):
        cnts, ts = carry
        j0 = ci * _L
        px16 = xs[pl.ds(j0, _L)]
        py16 = ys[pl.ds(j0, _L)]
        pz16 = zs[pl.ds(j0, _L)]
        psq16 = (px16 * px16 + py16 * py16) + pz16 * pz16
        cnts = list(cnts)
        for i in range(_L):
            px = jnp.full((_L,), px16[i])
            py = jnp.full((_L,), py16[i])
            pz = jnp.full((_L,), pz16[i])
            psq = jnp.full((_L,), psq16[i])
            jv = jnp.full((_L,), 1, dtype=jnp.int32) * (j0 + i)
            for g in range(_NG):
                ndp = nx[g] * px + ny[g] * py + nz[g] * pz
                d = (nsq[g] + psq) - 2.0 * ndp
                kb = lax.bitcast_convert_type(d, jnp.int32)
                m = kb < ts[g]
                cidx = jnp.minimum(cnts[g], _CAP - 1)
                fidx = cidx * _ROWW + colg[g]
                plsc.store_scatter(bkey, [fidx], kb, mask=m)
                plsc.store_scatter(bidx, [fidx], jv, mask=m)
                cnts[g] = cnts[g] + jnp.where(m, one16, zero16)
        cnts = tuple(cnts)

        def check(carry):
            cnts, ts = carry
            m = cnts[0]
            for gg in range(1, _NG):
                m = jnp.maximum(m, cnts[gg])
            maxc = jnp.max(m)
            return lax.cond(maxc >= _CAP - _L, prune_all, lambda a: a,
                            (cnts, ts))

        return check((cnts, ts))

    init = (tuple(zero16 for _ in range(_NG)),
            tuple(jnp.full((_L,), _MAXFIN, dtype=jnp.int32) for _ in range(_NG)))
    cnts, ts = lax.fori_loop(0, _N // _L, chunk_step, init)
    cnts, ts = prune_all((cnts, ts))

    num = jnp.float32(0.0)
    den = jnp.float32(0.0)
    for g in range(_NG):
        cnt = cnts[g]
        cmax = jnp.max(cnt)

        def bit_step(b, prefix):
            probe = prefix + (jnp.int32(1) << (jnp.int32(30) - b))

            def row_cnt(k, c):
                valid = jnp.logical_and(bkey[pl.ds(k * _ROWW + g * _L, _L)] < probe, k < cnt)
                return c + jnp.where(valid, one16, zero16)

            c = lax.fori_loop(0, cmax, row_cnt, zero16)
            return jnp.where(c >= _K, prefix, probe)

        t50 = lax.fori_loop(0, 31, bit_step, zero16)

        def stats_row(k, cs):
            c_lt, c_eq = cs
            row = bkey[pl.ds(k * _ROWW + g * _L, _L)]
            valid = k < cnt
            c_lt = c_lt + jnp.where(jnp.logical_and(row < t50, valid), one16, zero16)
            c_eq = c_eq + jnp.where(jnp.logical_and(row == t50, valid), one16, zero16)
            return c_lt, c_eq

        c_lt, c_eq = lax.fori_loop(0, cmax, stats_row, (zero16, zero16))
        frac = (_K - c_lt).astype(jnp.float32) / jnp.maximum(c_eq, 1).astype(jnp.float32)

        def sum_row(k, acc):
            sx, sy, sz = acc
            row = bkey[pl.ds(k * _ROWW + g * _L, _L)]
            ri = bidx[pl.ds(k * _ROWW + g * _L, _L)]
            valid = k < cnt
            lt = jnp.logical_and(row < t50, valid)
            eq = jnp.logical_and(row == t50, valid)
            w = jnp.where(lt, 1.0, 0.0) + jnp.where(eq, frac, 0.0)
            ri = jnp.bitwise_and(ri, _N - 1)
            gx = plsc.load_gather(xs, [ri])
            gy = plsc.load_gather(ys, [ri])
            gz = plsc.load_gather(zs, [ri])
            return sx + w * gx, sy + w * gy, sz + w * gz

        fzero = jnp.zeros((_L,), dtype=jnp.float32)
        sx, sy, sz = lax.fori_loop(0, cmax, sum_row, (fzero, fzero, fzero))
        mx = sx * (1.0 / _K)
        my = sy * (1.0 / _K)
        mz = sz * (1.0 / _K)

        pv = ps[pl.ds(0, _L)]
        mtx = pv[0] * mx + pv[1] * my + pv[2] * mz + pv[3]
        mty = pv[4] * mx + pv[5] * my + pv[6] * mz + pv[7]
        mtz = pv[8] * mx + pv[9] * my + pv[10] * mz + pv[11]

        sl = pl.ds(base + g * _L, _L)
        corr = (jnp.abs(mtx - kw[0, sl]) + jnp.abs(mty - kw[1, sl])
                + jnp.abs(mtz - kw[2, sl]))
        oww = ow[sl]
        num = num + jnp.sum(oww * corr)
        den = den + jnp.sum(oww)

    lane = lax.iota(jnp.int32, _L)
    outv[...] = jnp.where(lane == 0, num, jnp.where(lane == 1, den, 0.0))
    pltpu.sync_copy(outv, out_h.at[wid])


_sc_call = functools.partial(
    pl.kernel,
    out_type=jax.ShapeDtypeStruct((_NW, _L), jnp.float32),
    mesh=plsc.VectorSubcoreMesh(core_axis_name="c", subcore_axis_name="s",
                                num_cores=_NC, num_subcores=_NS),
    compiler_params=pltpu.CompilerParams(needs_layout_passes=False),
    scratch_types=[
        pltpu.VMEM((_N,), jnp.float32),
        pltpu.VMEM((_N,), jnp.float32),
        pltpu.VMEM((_N,), jnp.float32),
        pltpu.VMEM((3, _M), jnp.float32),
        pltpu.VMEM((3, _M), jnp.float32),
        pltpu.VMEM((_L,), jnp.float32),
        pltpu.VMEM((_M,), jnp.float32),
        pltpu.VMEM((_CAP * _NG * _L,), jnp.int32),
        pltpu.VMEM((_CAP * _NG * _L,), jnp.int32),
        pltpu.VMEM((_L,), jnp.float32),
    ],
)(_sc_body)


def _tc_kernel(pts_t_ref, nodes_ref, kpw_ref, pose_ref, ow_ref, out_ref, acc_ref):
    b = pl.program_id(0)
    c = pl.program_id(1)

    @pl.when(jnp.logical_and(b == 0, c == 0))
    def _init():
        acc_ref[0] = 0.0
        acc_ref[1] = 0.0

    pts_t = pts_t_ref[0]
    nodes = nodes_ref[0]

    ndp = lax.dot_general(nodes, pts_t, (((1,), (0,)), ((), ())),
                          preferred_element_type=jnp.float32)
    psq = jnp.sum(pts_t * pts_t, axis=0, keepdims=True)
    nsq = jnp.sum(nodes * nodes, axis=1, keepdims=True)
    d = (nsq + psq) - 2.0 * ndp

    bits = lax.bitcast_convert_type(d, jnp.int32)
    skey = jnp.where(bits >= 0, bits, bits ^ jnp.int32(0x7FFFFFFF))

    def search_step(i, prefix):
        bit = (jnp.uint32(31) - i.astype(jnp.uint32))
        probe = prefix + (jnp.uint32(1) << bit)
        sprobe = lax.bitcast_convert_type(probe ^ jnp.uint32(0x80000000), jnp.int32)
        cnt = jnp.sum((skey < sprobe[:, None]).astype(jnp.int32), axis=1)
        return jnp.where(cnt >= _K, prefix, probe)

    prefix0 = jnp.zeros((_CHUNK,), dtype=jnp.uint32)
    t_u = lax.fori_loop(0, 32, search_step, prefix0)
    t_s = lax.bitcast_convert_type(t_u ^ jnp.uint32(0x80000000), jnp.int32)

    lt = skey < t_s[:, None]
    eq = skey == t_s[:, None]
    cnt_lt = jnp.sum(lt.astype(jnp.int32), axis=1)
    t_cnt = jnp.sum(eq.astype(jnp.int32), axis=1)
    r = (_K - cnt_lt).astype(jnp.float32)
    frac = r / t_cnt.astype(jnp.float32)
    w = lt.astype(jnp.float32) + frac[:, None] * eq.astype(jnp.float32)

    sums = lax.dot_general(w, pts_t, (((1,), (1,)), ((), ())),
                           preferred_element_type=jnp.float32)
    mean = sums * (1.0 / _K)

    pose = pose_ref[0]
    rot = pose[:3, :3]
    trans = pose[:3, 3]
    mt = lax.dot_general(mean, rot, (((1,), (1,)), ((), ())),
                         preferred_element_type=jnp.float32) + trans[None, :]
    corr = jnp.sum(jnp.abs(mt - kpw_ref[0]), axis=1)
    ow = ow_ref[0, 0]

    acc_ref[0] += jnp.sum(ow * corr)
    acc_ref[1] += jnp.sum(ow)

    @pl.when(jnp.logical_and(b == _B - 1, c == _C - 1))
    def _fin():
        out_ref[...] = jnp.stack([acc_ref[0], acc_ref[1]]).reshape(1, 2)


def _tc_call(pts_t, nodes, kpw, pose, ow3):
    return pl.pallas_call(
        _tc_kernel,
        grid=(_B, _C),
        in_specs=[
            pl.BlockSpec((1, 3, _N), lambda b, c: (b, 0, 0)),
            pl.BlockSpec((1, _CHUNK, 3), lambda b, c: (b, c, 0)),
            pl.BlockSpec((1, _CHUNK, 3), lambda b, c: (b, c, 0)),
            pl.BlockSpec((1, 4, 4), lambda b, c: (b, 0, 0)),
            pl.BlockSpec((1, 1, _CHUNK), lambda b, c: (b * _C + c, 0, 0)),
        ],
        out_specs=pl.BlockSpec((1, 2), lambda b, c: (0, 0)),
        out_shape=jax.ShapeDtypeStruct((1, 2), jnp.float32),
        scratch_shapes=[pltpu.SMEM((2,), jnp.float32)],
    )(pts_t, nodes, kpw, pose, ow3)


@jax.jit
def kernel(pts_before, kp_before, kp_warped_pred, pose_gt, overlap_weights):
    xs = pts_before[:, :, 0]
    ys = pts_before[:, :, 1]
    zs = pts_before[:, :, 2]
    nd = jnp.swapaxes(kp_before, 1, 2)
    kw = jnp.swapaxes(kp_warped_pred, 1, 2)
    ps = pose_gt.reshape(_B, 16)
    parts = _sc_call(xs, ys, zs, nd, kw, ps, overlap_weights)

    pts_t = jnp.swapaxes(pts_before, 1, 2)
    ow3 = overlap_weights[:, _MSC:].reshape(_B * _C, 1, _CHUNK)
    tc = _tc_call(pts_t, kp_before[:, _MSC:, :], kp_warped_pred[:, _MSC:, :],
                  pose_gt, ow3)

    num = jnp.sum(parts[:, 0]) + tc[0, 0]
    den = jnp.sum(parts[:, 1]) + tc[0, 1]
    return num / jnp.maximum(den, _EPS)


# speedup vs baseline: 27.9612x; 1.3088x over previous
import functools

import jax
import jax.numpy as jnp
from jax import lax
from jax.experimental import pallas as pl
from jax.experimental.pallas import tpu as pltpu
from jax.experimental.pallas import tpu_sc as plsc

_EPS = 1e-06
_K = 50
_B = 4
_M = 512
_N = 16384
_L = 16
_NC = 2
_NS = 16
_NW = _NC * _NS
_TPB = _NW // _B
_MSC = 256
_NPT = _MSC // _TPB
_NG = _NPT // _L
_MTC = _M - _MSC
_CHUNK = 128
_C = _MTC // _CHUNK
_CAP = 256
_KCHK = 32
_MAXFIN = 0x7F7FFFFF


def _sc_body(xs_h, ys_h, zs_h, nd_h, kw_h, ps_h, ow_h, out_h,
             xs, ys, zs, nd, kw, ps, ow, bkey, bidx, outv):
    cid = lax.axis_index("c")
    sid = lax.axis_index("s")
    wid = sid * _NC + cid
    batch = wid // _TPB
    s8 = wid % _TPB

    pltpu.sync_copy(xs_h.at[batch], xs)
    pltpu.sync_copy(ys_h.at[batch], ys)
    pltpu.sync_copy(zs_h.at[batch], zs)
    pltpu.sync_copy(nd_h.at[batch], nd)
    pltpu.sync_copy(kw_h.at[batch], kw)
    pltpu.sync_copy(ps_h.at[batch], ps)
    pltpu.sync_copy(ow_h.at[batch], ow)

    col = lax.iota(jnp.int32, _L)
    base = s8 * _NPT

    nx = [nd[0, pl.ds(base + g * _L, _L)] for g in range(_NG)]
    ny = [nd[1, pl.ds(base + g * _L, _L)] for g in range(_NG)]
    nz = [nd[2, pl.ds(base + g * _L, _L)] for g in range(_NG)]
    colg = [col + g * _L for g in range(_NG)]
    _ROWW = _NG * _L

    zero16 = jnp.zeros((_L,), dtype=jnp.int32)
    one16 = jnp.full((_L,), 1, dtype=jnp.int32)

    def prune_group(g, cnt, t_cur):
        cmax = jnp.max(cnt)

        def do(args):
            cnt, t_cur = args
            f = bkey[pl.ds(g * _L, _L)]
            fmin = f
            for k in range(1, _K):
                row = bkey[pl.ds(k * _ROWW + g * _L, _L)]
                f = jnp.maximum(f, row)
                fmin = jnp.minimum(fmin, row)

            def ladder(lo, hi):
                step = lax.shift_right_arithmetic(hi - lo, 3)
                probes = [lo + i * step for i in range(1, 8)]

                def count_row(k, cs):
                    row = bkey[pl.ds(k * _ROWW + g * _L, _L)]
                    valid = k < cnt
                    return tuple(
                        c + jnp.where(jnp.logical_and(row < p, valid), one16, zero16)
                        for c, p in zip(cs, probes))

                counts = lax.fori_loop(0, cmax, count_row,
                                       tuple(zero16 for _ in probes))
                t_hi = hi
                for c, p in zip(reversed(counts), reversed(probes)):
                    t_hi = jnp.where(c >= _K, p, t_hi)
                t_lo = lo
                for c, p in zip(counts, probes):
                    t_lo = jnp.where(c < _K, p, t_lo)
                return t_hi, t_lo

            t1, lo1 = ladder(fmin, f)
            t_new, _ = ladder(lo1, t1)
            t_new = jnp.where(cnt >= _K, t_new, t_cur)

            def compact_row(k, ncnt):
                rk = bkey[pl.ds(k * _ROWW + g * _L, _L)]
                ri = bidx[pl.ds(k * _ROWW + g * _L, _L)]
                keep = jnp.logical_and(rk <= t_new, k < cnt)
                nc = jnp.minimum(ncnt, _CAP - 1) * _ROWW + colg[g]
                plsc.store_scatter(bkey, [nc], rk, mask=keep)
                plsc.store_scatter(bidx, [nc], ri, mask=keep)
                return ncnt + jnp.where(keep, one16, zero16)

            new_cnt = lax.fori_loop(0, cmax, compact_row, zero16)
            return new_cnt, t_new

        return lax.cond(cmax >= _K, do, lambda a: a, (cnt, t_cur))

    def prune_all(carry):
        cnts, ts = carry
        new = [prune_group(g, cnts[g], ts[g]) for g in range(_NG)]
        return tuple(n[0] for n in new), tuple(n[1] for n in new)

    nsq = [(nx[g] * nx[g] + ny[g] * ny[g]) + nz[g] * nz[g] for g in range(_NG)]

    def chunk_step(ci, carry):
        cnts, ts = carry
        j0 = ci * _L
        px16 = xs[pl.ds(j0, _L)]
        py16 = ys[pl.ds(j0, _L)]
        pz16 = zs[pl.ds(j0, _L)]
        psq16 = (px16 * px16 + py16 * py16) + pz16 * pz16
        cnts = list(cnts)
        for i in range(_L):
            px = jnp.full((_L,), px16[i])
            py = jnp.full((_L,), py16[i])
            pz = jnp.full((_L,), pz16[i])
            psq = jnp.full((_L,), psq16[i])
            jv = jnp.full((_L,), 1, dtype=jnp.int32) * (j0 + i)
            for g in range(_NG):
                ndp = nx[g] * px + ny[g] * py + nz[g] * pz
                d = (nsq[g] + psq) - 2.0 * ndp
                kb = lax.bitcast_convert_type(d, jnp.int32)
                m = kb < ts[g]
                cidx = jnp.minimum(cnts[g], _CAP - 1)
                fidx = cidx * _ROWW + colg[g]
                plsc.store_scatter(bkey, [fidx], kb, mask=m)
                plsc.store_scatter(bidx, [fidx], jv, mask=m)
                cnts[g] = cnts[g] + jnp.where(m, one16, zero16)
        cnts = tuple(cnts)

        def check(carry):
            cnts, ts = carry
            m = cnts[0]
            for gg in range(1, _NG):
                m = jnp.maximum(m, cnts[gg])
            maxc = jnp.max(m)
            return lax.cond(maxc >= _CAP - _L, prune_all, lambda a: a,
                            (cnts, ts))

        return check((cnts, ts))

    init = (tuple(zero16 for _ in range(_NG)),
            tuple(jnp.full((_L,), _MAXFIN, dtype=jnp.int32) for _ in range(_NG)))
    cnts, ts = lax.fori_loop(0, _N // _L, chunk_step, init)
    cnts, ts = prune_all((cnts, ts))

    num = jnp.float32(0.0)
    den = jnp.float32(0.0)
    for g in range(_NG):
        cnt = cnts[g]
        cmax = jnp.max(cnt)

        def bit_step(b, prefix):
            probe = prefix + (jnp.int32(1) << (jnp.int32(30) - b))

            def row_cnt(k, c):
                valid = jnp.logical_and(bkey[pl.ds(k * _ROWW + g * _L, _L)] < probe, k < cnt)
                return c + jnp.where(valid, one16, zero16)

            c = lax.fori_loop(0, cmax, row_cnt, zero16)
            return jnp.where(c >= _K, prefix, probe)

        t50 = lax.fori_loop(0, 31, bit_step, zero16)

        def stats_row(k, cs):
            c_lt, c_eq = cs
            row = bkey[pl.ds(k * _ROWW + g * _L, _L)]
            valid = k < cnt
            c_lt = c_lt + jnp.where(jnp.logical_and(row < t50, valid), one16, zero16)
            c_eq = c_eq + jnp.where(jnp.logical_and(row == t50, valid), one16, zero16)
            return c_lt, c_eq

        c_lt, c_eq = lax.fori_loop(0, cmax, stats_row, (zero16, zero16))
        frac = (_K - c_lt).astype(jnp.float32) / jnp.maximum(c_eq, 1).astype(jnp.float32)

        def sum_row(k, acc):
            sx, sy, sz = acc
            row = bkey[pl.ds(k * _ROWW + g * _L, _L)]
            ri = bidx[pl.ds(k * _ROWW + g * _L, _L)]
            valid = k < cnt
            lt = jnp.logical_and(row < t50, valid)
            eq = jnp.logical_and(row == t50, valid)
            w = jnp.where(lt, 1.0, 0.0) + jnp.where(eq, frac, 0.0)
            ri = jnp.bitwise_and(ri, _N - 1)
            gx = plsc.load_gather(xs, [ri])
            gy = plsc.load_gather(ys, [ri])
            gz = plsc.load_gather(zs, [ri])
            return sx + w * gx, sy + w * gy, sz + w * gz

        fzero = jnp.zeros((_L,), dtype=jnp.float32)
        sx, sy, sz = lax.fori_loop(0, cmax, sum_row, (fzero, fzero, fzero))
        mx = sx * (1.0 / _K)
        my = sy * (1.0 / _K)
        mz = sz * (1.0 / _K)

        pv = ps[pl.ds(0, _L)]
        mtx = pv[0] * mx + pv[1] * my + pv[2] * mz + pv[3]
        mty = pv[4] * mx + pv[5] * my + pv[6] * mz + pv[7]
        mtz = pv[8] * mx + pv[9] * my + pv[10] * mz + pv[11]

        sl = pl.ds(base + g * _L, _L)
        corr = (jnp.abs(mtx - kw[0, sl]) + jnp.abs(mty - kw[1, sl])
                + jnp.abs(mtz - kw[2, sl]))
        oww = ow[sl]
        num = num + jnp.sum(oww * corr)
        den = den + jnp.sum(oww)

    lane = lax.iota(jnp.int32, _L)
    outv[...] = jnp.where(lane == 0, num, jnp.where(lane == 1, den, 0.0))
    pltpu.sync_copy(outv, out_h.at[wid])


_sc_call = functools.partial(
    pl.kernel,
    out_type=jax.ShapeDtypeStruct((_NW, _L), jnp.float32),
    mesh=plsc.VectorSubcoreMesh(core_axis_name="c", subcore_axis_name="s",
                                num_cores=_NC, num_subcores=_NS),
    compiler_params=pltpu.CompilerParams(needs_layout_passes=False),
    scratch_types=[
        pltpu.VMEM((_N,), jnp.float32),
        pltpu.VMEM((_N,), jnp.float32),
        pltpu.VMEM((_N,), jnp.float32),
        pltpu.VMEM((3, _M), jnp.float32),
        pltpu.VMEM((3, _M), jnp.float32),
        pltpu.VMEM((_L,), jnp.float32),
        pltpu.VMEM((_M,), jnp.float32),
        pltpu.VMEM((_CAP * _NG * _L,), jnp.int32),
        pltpu.VMEM((_CAP * _NG * _L,), jnp.int32),
        pltpu.VMEM((_L,), jnp.float32),
    ],
)(_sc_body)


def _tc_kernel(pts_t_ref, nodes_ref, kpw_ref, pose_ref, ow_ref, out_ref, acc_ref):
    b = pl.program_id(0)
    c = pl.program_id(1)

    @pl.when(jnp.logical_and(b == 0, c == 0))
    def _init():
        acc_ref[0] = 0.0
        acc_ref[1] = 0.0

    pts_t = pts_t_ref[0]
    nodes = nodes_ref[0]

    ndp = lax.dot_general(nodes, pts_t, (((1,), (0,)), ((), ())),
                          preferred_element_type=jnp.float32)
    psq = jnp.sum(pts_t * pts_t, axis=0, keepdims=True)
    nsq = jnp.sum(nodes * nodes, axis=1, keepdims=True)
    d = (nsq + psq) - 2.0 * ndp

    bits = lax.bitcast_convert_type(d, jnp.int32)
    skey = jnp.where(bits >= 0, bits, bits ^ jnp.int32(0x7FFFFFFF))
    qkey = lax.shift_right_arithmetic(skey, 10)

    def search_step(i, prefix):
        bit = (jnp.int32(21) - i)
        probe = prefix + (jnp.int32(1) << bit)
        cnt = jnp.sum((qkey < probe[:, None]).astype(jnp.int32), axis=1)
        return jnp.where(cnt >= _K, prefix, probe)

    prefix0 = jnp.zeros((_CHUNK,), dtype=jnp.int32)
    t_q = lax.fori_loop(0, 22, search_step, prefix0)

    lt = qkey < t_q[:, None]
    eq = qkey == t_q[:, None]
    cnt_lt = jnp.sum(lt.astype(jnp.int32), axis=1)
    t_cnt = jnp.sum(eq.astype(jnp.int32), axis=1)
    r = (_K - cnt_lt).astype(jnp.float32)
    frac = r / t_cnt.astype(jnp.float32)
    w = lt.astype(jnp.float32) + frac[:, None] * eq.astype(jnp.float32)

    sums = lax.dot_general(w, pts_t, (((1,), (1,)), ((), ())),
                           preferred_element_type=jnp.float32)
    mean = sums * (1.0 / _K)

    pose = pose_ref[0]
    rot = pose[:3, :3]
    trans = pose[:3, 3]
    mt = lax.dot_general(mean, rot, (((1,), (1,)), ((), ())),
                         preferred_element_type=jnp.float32) + trans[None, :]
    corr = jnp.sum(jnp.abs(mt - kpw_ref[0]), axis=1)
    ow = ow_ref[0, 0]

    acc_ref[0] += jnp.sum(ow * corr)
    acc_ref[1] += jnp.sum(ow)

    @pl.when(jnp.logical_and(b == _B - 1, c == _C - 1))
    def _fin():
        out_ref[...] = jnp.stack([acc_ref[0], acc_ref[1]]).reshape(1, 2)


def _tc_call(pts_t, nodes, kpw, pose, ow3):
    return pl.pallas_call(
        _tc_kernel,
        grid=(_B, _C),
        in_specs=[
            pl.BlockSpec((1, 3, _N), lambda b, c: (b, 0, 0)),
            pl.BlockSpec((1, _CHUNK, 3), lambda b, c: (b, c, 0)),
            pl.BlockSpec((1, _CHUNK, 3), lambda b, c: (b, c, 0)),
            pl.BlockSpec((1, 4, 4), lambda b, c: (b, 0, 0)),
            pl.BlockSpec((1, 1, _CHUNK), lambda b, c: (b * _C + c, 0, 0)),
        ],
        out_specs=pl.BlockSpec((1, 2), lambda b, c: (0, 0)),
        out_shape=jax.ShapeDtypeStruct((1, 2), jnp.float32),
        scratch_shapes=[pltpu.SMEM((2,), jnp.float32)],
    )(pts_t, nodes, kpw, pose, ow3)


@jax.jit
def kernel(pts_before, kp_before, kp_warped_pred, pose_gt, overlap_weights):
    xs = pts_before[:, :, 0]
    ys = pts_before[:, :, 1]
    zs = pts_before[:, :, 2]
    nd = jnp.swapaxes(kp_before, 1, 2)
    kw = jnp.swapaxes(kp_warped_pred, 1, 2)
    ps = pose_gt.reshape(_B, 16)
    parts = _sc_call(xs, ys, zs, nd, kw, ps, overlap_weights)

    pts_t = jnp.swapaxes(pts_before, 1, 2)
    ow3 = overlap_weights[:, _MSC:].reshape(_B * _C, 1, _CHUNK)
    tc = _tc_call(pts_t, kp_before[:, _MSC:, :], kp_warped_pred[:, _MSC:, :],
                  pose_gt, ow3)

    num = jnp.sum(parts[:, 0]) + tc[0, 0]
    den = jnp.sum(parts[:, 1]) + tc[0, 1]
    return num / jnp.maximum(den, _EPS)


# speedup vs baseline: 29.7710x; 1.0647x over previous
import functools

import jax
import jax.numpy as jnp
from jax import lax
from jax.experimental import pallas as pl
from jax.experimental.pallas import tpu as pltpu
from jax.experimental.pallas import tpu_sc as plsc

_EPS = 1e-06
_K = 50
_B = 4
_M = 512
_N = 16384
_L = 16
_NC = 2
_NS = 16
_NW = _NC * _NS
_TPB = _NW // _B
_MSC = 256
_NPT = _MSC // _TPB
_NG = _NPT // _L
_MTC = _M - _MSC
_CHUNK = 128
_C = _MTC // _CHUNK
_CAP = 256
_KCHK = 32
_MAXFIN = 0x7F7FFFFF


def _sc_body(xs_h, ys_h, zs_h, nd_h, kw_h, ps_h, ow_h, out_h,
             xs, ys, zs, nd, kw, ps, ow, bkey, bidx, outv):
    cid = lax.axis_index("c")
    sid = lax.axis_index("s")
    wid = sid * _NC + cid
    batch = wid // _TPB
    s8 = wid % _TPB

    pltpu.sync_copy(xs_h.at[batch], xs)
    pltpu.sync_copy(ys_h.at[batch], ys)
    pltpu.sync_copy(zs_h.at[batch], zs)
    pltpu.sync_copy(nd_h.at[batch], nd)
    pltpu.sync_copy(kw_h.at[batch], kw)
    pltpu.sync_copy(ps_h.at[batch], ps)
    pltpu.sync_copy(ow_h.at[batch], ow)

    col = lax.iota(jnp.int32, _L)
    base = s8 * _NPT

    nx = [nd[0, pl.ds(base + g * _L, _L)] for g in range(_NG)]
    ny = [nd[1, pl.ds(base + g * _L, _L)] for g in range(_NG)]
    nz = [nd[2, pl.ds(base + g * _L, _L)] for g in range(_NG)]
    colg = [col + g * _L for g in range(_NG)]
    _ROWW = _NG * _L
    _ROWSH = _ROWW.bit_length() - 1
    assert _ROWW == 1 << _ROWSH

    zero16 = jnp.zeros((_L,), dtype=jnp.int32)
    one16 = jnp.full((_L,), 1, dtype=jnp.int32)
    roww16 = jnp.full((_L,), _ROWW, dtype=jnp.int32)

    def prune_group(g, cntm, t_cur):
        cnt = lax.shift_right_arithmetic(cntm, _ROWSH)
        cmax = jnp.max(cnt)

        def do(args):
            cnt, t_cur = args
            del args
            f = bkey[pl.ds(g * _L, _L)]
            fmin = f
            for k in range(1, _K):
                row = bkey[pl.ds(k * _ROWW + g * _L, _L)]
                f = jnp.maximum(f, row)
                fmin = jnp.minimum(fmin, row)

            def ladder(lo, hi):
                step = lax.shift_right_arithmetic(hi - lo, 3)
                probes = [lo + i * step for i in range(1, 8)]

                def count_row(k, cs):
                    row = bkey[pl.ds(k * _ROWW + g * _L, _L)]
                    valid = k < cnt
                    return tuple(
                        c + jnp.where(jnp.logical_and(row < p, valid), one16, zero16)
                        for c, p in zip(cs, probes))

                counts = lax.fori_loop(0, cmax, count_row,
                                       tuple(zero16 for _ in probes))
                t_hi = hi
                for c, p in zip(reversed(counts), reversed(probes)):
                    t_hi = jnp.where(c >= _K, p, t_hi)
                t_lo = lo
                for c, p in zip(counts, probes):
                    t_lo = jnp.where(c < _K, p, t_lo)
                return t_hi, t_lo

            t1, lo1 = ladder(fmin, f)
            t_new, _ = ladder(lo1, t1)
            t_new = jnp.where(cnt >= _K, t_new, t_cur)

            def compact_row(k, ncnt):
                rk = bkey[pl.ds(k * _ROWW + g * _L, _L)]
                ri = bidx[pl.ds(k * _ROWW + g * _L, _L)]
                keep = jnp.logical_and(rk <= t_new, k < cnt)
                nc = jnp.minimum(ncnt, _CAP - 1) * _ROWW + colg[g]
                plsc.store_scatter(bkey, [nc], rk, mask=keep)
                plsc.store_scatter(bidx, [nc], ri, mask=keep)
                return ncnt + jnp.where(keep, one16, zero16)

            new_cnt = lax.fori_loop(0, cmax, compact_row, zero16)
            return lax.shift_left(new_cnt, _ROWSH), t_new

        return lax.cond(cmax >= _K, do, lambda a: (lax.shift_left(a[0], _ROWSH), a[1]),
                        (cnt, t_cur))

    def prune_all(carry):
        cnts, ts = carry
        new = [prune_group(g, cnts[g], ts[g]) for g in range(_NG)]
        return tuple(n[0] for n in new), tuple(n[1] for n in new)

    nsq = [(nx[g] * nx[g] + ny[g] * ny[g]) + nz[g] * nz[g] for g in range(_NG)]

    def chunk_step(ci, carry):
        cnts, ts = carry
        j0 = ci * _L
        jvc = jnp.full((_L,), 1, dtype=jnp.int32) * j0
        px16 = xs[pl.ds(j0, _L)]
        py16 = ys[pl.ds(j0, _L)]
        pz16 = zs[pl.ds(j0, _L)]
        psq16 = (px16 * px16 + py16 * py16) + pz16 * pz16
        cnts = list(cnts)
        for i in range(_L):
            px = jnp.full((_L,), px16[i])
            py = jnp.full((_L,), py16[i])
            pz = jnp.full((_L,), pz16[i])
            psq = jnp.full((_L,), psq16[i])
            jv = jvc + i
            for g in range(_NG):
                ndp = nx[g] * px + ny[g] * py + nz[g] * pz
                d = (nsq[g] + psq) - 2.0 * ndp
                kb = lax.bitcast_convert_type(d, jnp.int32)
                m = kb < ts[g]
                fidx = cnts[g] + colg[g]
                plsc.store_scatter(bkey, [fidx], kb, mask=m)
                plsc.store_scatter(bidx, [fidx], jv, mask=m)
                cnts[g] = cnts[g] + jnp.where(m, roww16, zero16)
        cnts = tuple(cnts)

        def check(carry):
            cnts, ts = carry
            m = cnts[0]
            for gg in range(1, _NG):
                m = jnp.maximum(m, cnts[gg])
            maxc = jnp.max(m)
            cnts, ts = lax.cond(maxc >= (_CAP - _L) * _ROWW, prune_all,
                                lambda a: a, (cnts, ts))
            cnts = tuple(jnp.minimum(c, (_CAP - _L - 1) * _ROWW) for c in cnts)
            return cnts, ts

        return check((cnts, ts))

    init = (tuple(zero16 for _ in range(_NG)),
            tuple(jnp.full((_L,), _MAXFIN, dtype=jnp.int32) for _ in range(_NG)))
    cnts, ts = lax.fori_loop(0, _N // _L, chunk_step, init)
    cnts, ts = prune_all((cnts, ts))

    num = jnp.float32(0.0)
    den = jnp.float32(0.0)
    for g in range(_NG):
        cnt = lax.shift_right_arithmetic(cnts[g], _ROWSH)
        cmax = jnp.max(cnt)

        def bit_step(b, prefix):
            probe = prefix + (jnp.int32(1) << (jnp.int32(30) - b))

            def row_cnt(k, c):
                valid = jnp.logical_and(bkey[pl.ds(k * _ROWW + g * _L, _L)] < probe, k < cnt)
                return c + jnp.where(valid, one16, zero16)

            c = lax.fori_loop(0, cmax, row_cnt, zero16)
            return jnp.where(c >= _K, prefix, probe)

        t50 = lax.fori_loop(0, 31, bit_step, zero16)

        def stats_row(k, cs):
            c_lt, c_eq = cs
            row = bkey[pl.ds(k * _ROWW + g * _L, _L)]
            valid = k < cnt
            c_lt = c_lt + jnp.where(jnp.logical_and(row < t50, valid), one16, zero16)
            c_eq = c_eq + jnp.where(jnp.logical_and(row == t50, valid), one16, zero16)
            return c_lt, c_eq

        c_lt, c_eq = lax.fori_loop(0, cmax, stats_row, (zero16, zero16))
        frac = (_K - c_lt).astype(jnp.float32) / jnp.maximum(c_eq, 1).astype(jnp.float32)

        def sum_row(k, acc):
            sx, sy, sz = acc
            row = bkey[pl.ds(k * _ROWW + g * _L, _L)]
            ri = bidx[pl.ds(k * _ROWW + g * _L, _L)]
            valid = k < cnt
            lt = jnp.logical_and(row < t50, valid)
            eq = jnp.logical_and(row == t50, valid)
            w = jnp.where(lt, 1.0, 0.0) + jnp.where(eq, frac, 0.0)
            ri = jnp.bitwise_and(ri, _N - 1)
            gx = plsc.load_gather(xs, [ri])
            gy = plsc.load_gather(ys, [ri])
            gz = plsc.load_gather(zs, [ri])
            return sx + w * gx, sy + w * gy, sz + w * gz

        fzero = jnp.zeros((_L,), dtype=jnp.float32)
        sx, sy, sz = lax.fori_loop(0, cmax, sum_row, (fzero, fzero, fzero))
        mx = sx * (1.0 / _K)
        my = sy * (1.0 / _K)
        mz = sz * (1.0 / _K)

        pv = ps[pl.ds(0, _L)]
        mtx = pv[0] * mx + pv[1] * my + pv[2] * mz + pv[3]
        mty = pv[4] * mx + pv[5] * my + pv[6] * mz + pv[7]
        mtz = pv[8] * mx + pv[9] * my + pv[10] * mz + pv[11]

        sl = pl.ds(base + g * _L, _L)
        corr = (jnp.abs(mtx - kw[0, sl]) + jnp.abs(mty - kw[1, sl])
                + jnp.abs(mtz - kw[2, sl]))
        oww = ow[sl]
        num = num + jnp.sum(oww * corr)
        den = den + jnp.sum(oww)

    lane = lax.iota(jnp.int32, _L)
    outv[...] = jnp.where(lane == 0, num, jnp.where(lane == 1, den, 0.0))
    pltpu.sync_copy(outv, out_h.at[wid])


_sc_call = functools.partial(
    pl.kernel,
    out_type=jax.ShapeDtypeStruct((_NW, _L), jnp.float32),
    mesh=plsc.VectorSubcoreMesh(core_axis_name="c", subcore_axis_name="s",
                                num_cores=_NC, num_subcores=_NS),
    compiler_params=pltpu.CompilerParams(needs_layout_passes=False),
    scratch_types=[
        pltpu.VMEM((_N,), jnp.float32),
        pltpu.VMEM((_N,), jnp.float32),
        pltpu.VMEM((_N,), jnp.float32),
        pltpu.VMEM((3, _M), jnp.float32),
        pltpu.VMEM((3, _M), jnp.float32),
        pltpu.VMEM((_L,), jnp.float32),
        pltpu.VMEM((_M,), jnp.float32),
        pltpu.VMEM((_CAP * _NG * _L,), jnp.int32),
        pltpu.VMEM((_CAP * _NG * _L,), jnp.int32),
        pltpu.VMEM((_L,), jnp.float32),
    ],
)(_sc_body)


def _tc_kernel(pts_t_ref, nodes_ref, kpw_ref, pose_ref, ow_ref, out_ref, acc_ref):
    b = pl.program_id(0)
    c = pl.program_id(1)

    @pl.when(jnp.logical_and(b == 0, c == 0))
    def _init():
        acc_ref[0] = 0.0
        acc_ref[1] = 0.0

    pts_t = pts_t_ref[0]
    nodes = nodes_ref[0]

    ndp = lax.dot_general(nodes, pts_t, (((1,), (0,)), ((), ())),
                          preferred_element_type=jnp.float32)
    psq = jnp.sum(pts_t * pts_t, axis=0, keepdims=True)
    nsq = jnp.sum(nodes * nodes, axis=1, keepdims=True)
    d = (nsq + psq) - 2.0 * ndp

    bits = lax.bitcast_convert_type(d, jnp.int32)
    skey = jnp.where(bits >= 0, bits, bits ^ jnp.int32(0x7FFFFFFF))
    qkey = lax.shift_right_arithmetic(skey, 12)

    def search_step(i, prefix):
        bit = (jnp.int32(19) - i)
        probe = prefix + (jnp.int32(1) << bit)
        cnt = jnp.sum((qkey < probe[:, None]).astype(jnp.int32), axis=1)
        return jnp.where(cnt >= _K, prefix, probe)

    prefix0 = jnp.zeros((_CHUNK,), dtype=jnp.int32)
    t_q = lax.fori_loop(0, 20, search_step, prefix0)

    lt = qkey < t_q[:, None]
    eq = qkey == t_q[:, None]
    cnt_lt = jnp.sum(lt.astype(jnp.int32), axis=1)
    t_cnt = jnp.sum(eq.astype(jnp.int32), axis=1)
    r = (_K - cnt_lt).astype(jnp.float32)
    frac = r / t_cnt.astype(jnp.float32)
    w = lt.astype(jnp.float32) + frac[:, None] * eq.astype(jnp.float32)

    sums = lax.dot_general(w, pts_t, (((1,), (1,)), ((), ())),
                           preferred_element_type=jnp.float32)
    mean = sums * (1.0 / _K)

    pose = pose_ref[0]
    rot = pose[:3, :3]
    trans = pose[:3, 3]
    mt = lax.dot_general(mean, rot, (((1,), (1,)), ((), ())),
                         preferred_element_type=jnp.float32) + trans[None, :]
    corr = jnp.sum(jnp.abs(mt - kpw_ref[0]), axis=1)
    ow = ow_ref[0, 0]

    acc_ref[0] += jnp.sum(ow * corr)
    acc_ref[1] += jnp.sum(ow)

    @pl.when(jnp.logical_and(b == _B - 1, c == _C - 1))
    def _fin():
        out_ref[...] = jnp.stack([acc_ref[0], acc_ref[1]]).reshape(1, 2)


def _tc_call(pts_t, nodes, kpw, pose, ow3):
    return pl.pallas_call(
        _tc_kernel,
        grid=(_B, _C),
        in_specs=[
            pl.BlockSpec((1, 3, _N), lambda b, c: (b, 0, 0)),
            pl.BlockSpec((1, _CHUNK, 3), lambda b, c: (b, c, 0)),
            pl.BlockSpec((1, _CHUNK, 3), lambda b, c: (b, c, 0)),
            pl.BlockSpec((1, 4, 4), lambda b, c: (b, 0, 0)),
            pl.BlockSpec((1, 1, _CHUNK), lambda b, c: (b * _C + c, 0, 0)),
        ],
        out_specs=pl.BlockSpec((1, 2), lambda b, c: (0, 0)),
        out_shape=jax.ShapeDtypeStruct((1, 2), jnp.float32),
        scratch_shapes=[pltpu.SMEM((2,), jnp.float32)],
    )(pts_t, nodes, kpw, pose, ow3)


@jax.jit
def kernel(pts_before, kp_before, kp_warped_pred, pose_gt, overlap_weights):
    xs = pts_before[:, :, 0]
    ys = pts_before[:, :, 1]
    zs = pts_before[:, :, 2]
    nd = jnp.swapaxes(kp_before, 1, 2)
    kw = jnp.swapaxes(kp_warped_pred, 1, 2)
    ps = pose_gt.reshape(_B, 16)
    parts = _sc_call(xs, ys, zs, nd, kw, ps, overlap_weights)

    pts_t = jnp.swapaxes(pts_before, 1, 2)
    ow3 = overlap_weights[:, _MSC:].reshape(_B * _C, 1, _CHUNK)
    tc = _tc_call(pts_t, kp_before[:, _MSC:, :], kp_warped_pred[:, _MSC:, :],
                  pose_gt, ow3)

    num = jnp.sum(parts[:, 0]) + tc[0, 0]
    den = jnp.sum(parts[:, 1]) + tc[0, 1]
    return num / jnp.maximum(den, _EPS)
